# baseline (speedup 1.0000x reference)
import functools

import jax
import jax.numpy as jnp
from jax import lax
from jax.experimental import pallas as pl
from jax.experimental.pallas import tpu as pltpu
from jax.experimental.pallas import tpu_sc as plsc

N = 10000
D = 128
H = 64
E = 320000

NC = 2
NS = 16
NW = NC * NS
C = 128
NCHUNK = E // C
NPAIR_TOT = NCHUNK // 2
PBASE = NPAIR_TOT // NW
PEXTRA = NPAIR_TOT - PBASE * NW
CHMAX = 2 * (PBASE + 1)
RPT = 640
NRP = NS * RPT

_mesh = plsc.VectorSubcoreMesh(
    core_axis_name="c", subcore_axis_name="s", num_cores=NC, num_subcores=NS
)

_sc_params = pltpu.CompilerParams(use_tc_tiling_on_sc=False)


def _load_idx_slab(ei3, which, cbase, wid, slab):
    pltpu.sync_copy(ei3.at[which, pl.ds(cbase, 2 * PBASE)],
                    slab.at[pl.ds(0, 2 * PBASE)])

    @pl.when(wid < PEXTRA)
    def _():
        pltpu.sync_copy(ei3.at[which, pl.ds(cbase + 2 * PBASE, 2)],
                        slab.at[pl.ds(2 * PBASE, 2)])


@functools.partial(
    pl.kernel,
    out_type=jax.ShapeDtypeStruct((NC, NRP), jnp.float32),
    mesh=_mesh,
    scratch_types=[
        pltpu.VMEM((CHMAX, C), jnp.int32),
        pltpu.VMEM((C,), jnp.float32),
        pltpu.VMEM((RPT,), jnp.float32),
        pltpu.VMEM_SHARED((NRP,), jnp.float32),
    ],
    compiler_params=_sc_params,
)
def _deg_kernel(ei3, out, idxs_v, ones_v, zb_v, acc):
    c = lax.axis_index("c")
    s = lax.axis_index("s")
    wid = c * NS + s
    npair = PBASE + jnp.where(wid < PEXTRA, 1, 0)
    cbase = 2 * (PBASE * wid + jnp.minimum(wid, PEXTRA))
    one = jnp.ones((16,), jnp.float32)
    zero = jnp.zeros((16,), jnp.float32)
    for q in range(C // 16):
        ones_v[pl.ds(q * 16, 16)] = one
    for q in range(RPT // 16):
        zb_v[pl.ds(q * 16, 16)] = zero
    _load_idx_slab(ei3, 1, cbase, wid, idxs_v)
    pltpu.sync_copy(zb_v, acc.at[pl.ds(s * RPT, RPT)])
    plsc.subcore_barrier()

    def body(j, carry):
        pltpu.sync_copy(ones_v, acc.at[idxs_v.at[j]], add=True)
        return carry

    lax.fori_loop(0, 2 * npair, body, 0)
    plsc.subcore_barrier()
    pltpu.sync_copy(acc.at[pl.ds(s * RPT, RPT)], out.at[c, pl.ds(s * RPT, RPT)])


@functools.partial(
    pl.kernel,
    out_type=jax.ShapeDtypeStruct((NC, NRP, H), jnp.float32),
    mesh=_mesh,
    scratch_types=[
        pltpu.VMEM((CHMAX, C), jnp.int32),
        pltpu.VMEM((CHMAX, C), jnp.int32),
    ] + [pltpu.VMEM((C, H), jnp.float32)] * 4 + [
        pltpu.VMEM((64, H), jnp.float32),
        pltpu.VMEM_SHARED((NRP, H), jnp.float32),
    ] + [pltpu.SemaphoreType.DMA] * 8,
    compiler_params=_sc_params,
)
def _agg_kernel(hp, ei3, out, srcs_v, dsts_v, r0, r1, r2, r3, zb_v, acc,
                g0, g1, g2, g3, s0, s1, s2, s3):
    c = lax.axis_index("c")
    s = lax.axis_index("s")
    wid = c * NS + s
    npair = PBASE + jnp.where(wid < PEXTRA, 1, 0)
    nch = 2 * npair
    cbase = 2 * (PBASE * wid + jnp.minimum(wid, PEXTRA))
    zero = jnp.zeros((16,), jnp.float32)
    for r in range(64):
        for q in range(H // 16):
            zb_v[r, pl.ds(q * 16, 16)] = zero
    _load_idx_slab(ei3, 0, cbase, wid, srcs_v)
    _load_idx_slab(ei3, 1, cbase, wid, dsts_v)
    for k in range(RPT // 64):
        pltpu.sync_copy(zb_v, acc.at[pl.ds(s * RPT + k * 64, 64)])
    plsc.subcore_barrier()

    def gather(j, rb, gb):
        pltpu.async_copy(hp.at[srcs_v.at[j]], rb, gb)

    def wait_gather(rb, gb):
        pltpu.make_async_copy(hp.at[srcs_v.at[0]], rb, gb).wait()

    def scatter(j, rb, sb):
        pltpu.async_copy(rb, acc.at[dsts_v.at[j]], sb, add=True)

    def wait_scatter(rb, sb):
        pltpu.make_async_copy(rb, acc.at[dsts_v.at[0]], sb).wait()

    gather(0, r0, g0)
    gather(1, r1, g1)

    def phase(i, j, ra0, ra1, ga0, ga1, sa0, sa1, rb0, rb1, gb0, gb1, sb0, sb1):
        @pl.when(i > 0)
        def _():
            wait_scatter(rb0, sb0)
            wait_scatter(rb1, sb1)

        @pl.when(j + 2 < nch)
        def _():
            gather(j + 2, rb0, gb0)

        @pl.when(j + 3 < nch)
        def _():
            gather(j + 3, rb1, gb1)

        wait_gather(ra0, ga0)
        scatter(j, ra0, sa0)
        wait_gather(ra1, ga1)
        scatter(j + 1, ra1, sa1)

    def body(i, carry):
        j = 2 * i

        @pl.when(i % 2 == 0)
        def _():
            phase(i, j, r0, r1, g0, g1, s0, s1, r2, r3, g2, g3, s2, s3)

        @pl.when(i % 2 == 1)
        def _():
            phase(i, j, r2, r3, g2, g3, s2, s3, r0, r1, g0, g1, s0, s1)

        return carry

    lax.fori_loop(0, npair, body, 0)

    @pl.when((npair - 1) % 2 == 0)
    def _():
        wait_scatter(r0, s0)
        wait_scatter(r1, s1)

    @pl.when((npair - 1) % 2 == 1)
    def _():
        wait_scatter(r2, s2)
        wait_scatter(r3, s3)

    plsc.subcore_barrier()
    pltpu.sync_copy(acc.at[pl.ds(s * RPT, RPT)], out.at[c, pl.ds(s * RPT, RPT)])


def _dinv_col(t):
    cols = [t[:, r:r + 1] for r in range(NRP // C)]
    return jnp.concatenate(cols, axis=0)[:N, :]


def _pre_body(degp_ref, x_ref, w1_ref, hp_ref, t_ref):
    d = degp_ref[0] + degp_ref[1] + 1.0
    dv = lax.rsqrt(d)
    t = jnp.transpose(dv)
    t_ref[...] = t
    h = jnp.dot(x_ref[...], w1_ref[...], preferred_element_type=jnp.float32)
    hp_ref[...] = h * _dinv_col(t)


_pre_call = pl.pallas_call(
    _pre_body,
    out_shape=(
        jax.ShapeDtypeStruct((N, H), jnp.float32),
        jax.ShapeDtypeStruct((C, NRP // C), jnp.float32),
    ),
)


def _mid_body(agg_ref, hp1_ref, t_ref, b1_ref, w2_ref, hp2_ref):
    dcol = _dinv_col(t_ref[...])
    a = agg_ref[0, :N, :] + agg_ref[1, :N, :]
    z1 = jnp.maximum(dcol * (a + hp1_ref[...]) + b1_ref[...], 0.0)
    hp2_ref[...] = jnp.dot(z1, w2_ref[...], preferred_element_type=jnp.float32) * dcol


_mid_call = pl.pallas_call(
    _mid_body,
    out_shape=jax.ShapeDtypeStruct((N, H), jnp.float32),
)


def _post_body(agg_ref, hp2_ref, t_ref, b2_ref, wl_ref, bl_ref, y_ref):
    dcol = _dinv_col(t_ref[...])
    a = agg_ref[0, :N, :] + agg_ref[1, :N, :]
    z2 = jnp.maximum(dcol * (a + hp2_ref[...]) + b2_ref[...], 0.0)
    y_ref[...] = jnp.dot(z2, wl_ref[...], preferred_element_type=jnp.float32) + bl_ref[...]


_post_call = pl.pallas_call(
    _post_body,
    out_shape=jax.ShapeDtypeStruct((N, 1), jnp.float32),
)


def kernel(x, edge_index, W1, b1, W2, b2, Wl, bl):
    ei3 = edge_index.reshape(2, NCHUNK, C)
    degp = _deg_kernel(ei3).reshape(NC, NRP // C, C)
    hp1, t = _pre_call(degp, x, W1)
    agg1 = _agg_kernel(hp1, ei3)
    hp2 = _mid_call(agg1, hp1, t, b1.reshape(1, H), W2)
    agg2 = _agg_kernel(hp2, ei3)
    y = _post_call(agg2, hp2, t, b2.reshape(1, H), Wl, bl.reshape(1, 1))
    return y


# speedup vs baseline: 1.0269x; 1.0269x over previous
import functools

import jax
import jax.numpy as jnp
from jax import lax
from jax.experimental import pallas as pl
from jax.experimental.pallas import tpu as pltpu
from jax.experimental.pallas import tpu_sc as plsc

N = 10000
D = 128
H = 64
E = 320000

NC = 2
NS = 16
NW = NC * NS
C = 128
NCHUNK = E // C
NPAIR_TOT = NCHUNK // 2
PBASE = NPAIR_TOT // NW
PEXTRA = NPAIR_TOT - PBASE * NW
CHMAX = 2 * (PBASE + 1)
RPT = 640
NRP = NS * RPT

_mesh = plsc.VectorSubcoreMesh(
    core_axis_name="c", subcore_axis_name="s", num_cores=NC, num_subcores=NS
)

_sc_params = pltpu.CompilerParams(use_tc_tiling_on_sc=False)


def _load_idx_slab(ei3, which, cbase, wid, slab):
    pltpu.sync_copy(ei3.at[which, pl.ds(cbase, 2 * PBASE)],
                    slab.at[pl.ds(0, 2 * PBASE)])

    @pl.when(wid < PEXTRA)
    def _():
        pltpu.sync_copy(ei3.at[which, pl.ds(cbase + 2 * PBASE, 2)],
                        slab.at[pl.ds(2 * PBASE, 2)])


@functools.partial(
    pl.kernel,
    out_type=jax.ShapeDtypeStruct((NC, NRP), jnp.float32),
    mesh=_mesh,
    scratch_types=[
        pltpu.VMEM((CHMAX, C), jnp.int32),
        pltpu.VMEM((C,), jnp.float32),
        pltpu.VMEM((RPT,), jnp.float32),
        pltpu.VMEM_SHARED((NRP,), jnp.float32),
        pltpu.SemaphoreType.DMA,
    ],
    compiler_params=_sc_params,
)
def _deg_kernel(ei3, out, idxs_v, ones_v, zb_v, acc, sem):
    c = lax.axis_index("c")
    s = lax.axis_index("s")
    wid = c * NS + s
    npair = PBASE + jnp.where(wid < PEXTRA, 1, 0)
    cbase = 2 * (PBASE * wid + jnp.minimum(wid, PEXTRA))
    one = jnp.ones((16,), jnp.float32)
    zero = jnp.zeros((16,), jnp.float32)
    for q in range(C // 16):
        ones_v[pl.ds(q * 16, 16)] = one
    for q in range(RPT // 16):
        zb_v[pl.ds(q * 16, 16)] = zero
    _load_idx_slab(ei3, 1, cbase, wid, idxs_v)
    pltpu.sync_copy(zb_v, acc.at[pl.ds(s * RPT, RPT)])
    plsc.subcore_barrier()

    def body(j, carry):
        pltpu.async_copy(ones_v, acc.at[idxs_v.at[j]], sem, add=True)
        return carry

    lax.fori_loop(0, 2 * npair, body, 0)

    def drain(j, carry):
        pltpu.make_async_copy(ones_v, acc.at[idxs_v.at[0]], sem).wait()
        return carry

    lax.fori_loop(0, 2 * npair, drain, 0)
    plsc.subcore_barrier()
    pltpu.sync_copy(acc.at[pl.ds(s * RPT, RPT)], out.at[c, pl.ds(s * RPT, RPT)])


@functools.partial(
    pl.kernel,
    out_type=jax.ShapeDtypeStruct((NC, NRP, H), jnp.float32),
    mesh=_mesh,
    scratch_types=[
        pltpu.VMEM((CHMAX, C), jnp.int32),
        pltpu.VMEM((CHMAX, C), jnp.int32),
    ] + [pltpu.VMEM((C, H), jnp.float32)] * 4 + [
        pltpu.VMEM((64, H), jnp.float32),
        pltpu.VMEM_SHARED((NRP, H), jnp.float32),
    ] + [pltpu.SemaphoreType.DMA] * 8,
    compiler_params=_sc_params,
)
def _agg_kernel(hp, ei3, out, srcs_v, dsts_v, r0, r1, r2, r3, zb_v, acc,
                g0, g1, g2, g3, s0, s1, s2, s3):
    c = lax.axis_index("c")
    s = lax.axis_index("s")
    wid = c * NS + s
    npair = PBASE + jnp.where(wid < PEXTRA, 1, 0)
    nch = 2 * npair
    cbase = 2 * (PBASE * wid + jnp.minimum(wid, PEXTRA))
    zero = jnp.zeros((16,), jnp.float32)
    for r in range(64):
        for q in range(H // 16):
            zb_v[r, pl.ds(q * 16, 16)] = zero
    _load_idx_slab(ei3, 0, cbase, wid, srcs_v)
    _load_idx_slab(ei3, 1, cbase, wid, dsts_v)
    for k in range(RPT // 64):
        pltpu.sync_copy(zb_v, acc.at[pl.ds(s * RPT + k * 64, 64)])
    plsc.subcore_barrier()

    def gather(j, rb, gb):
        pltpu.async_copy(hp.at[srcs_v.at[j]], rb, gb)

    def wait_gather(rb, gb):
        pltpu.make_async_copy(hp.at[srcs_v.at[0]], rb, gb).wait()

    def scatter(j, rb, sb):
        pltpu.async_copy(rb, acc.at[dsts_v.at[j]], sb, add=True)

    def wait_scatter(rb, sb):
        pltpu.make_async_copy(rb, acc.at[dsts_v.at[0]], sb).wait()

    gather(0, r0, g0)
    gather(1, r1, g1)

    def phase(i, j, ra0, ra1, ga0, ga1, sa0, sa1, rb0, rb1, gb0, gb1, sb0, sb1):
        @pl.when(i > 0)
        def _():
            wait_scatter(rb0, sb0)
            wait_scatter(rb1, sb1)

        @pl.when(j + 2 < nch)
        def _():
            gather(j + 2, rb0, gb0)

        @pl.when(j + 3 < nch)
        def _():
            gather(j + 3, rb1, gb1)

        wait_gather(ra0, ga0)
        scatter(j, ra0, sa0)
        wait_gather(ra1, ga1)
        scatter(j + 1, ra1, sa1)

    def body(i, carry):
        j = 2 * i

        @pl.when(i % 2 == 0)
        def _():
            phase(i, j, r0, r1, g0, g1, s0, s1, r2, r3, g2, g3, s2, s3)

        @pl.when(i % 2 == 1)
        def _():
            phase(i, j, r2, r3, g2, g3, s2, s3, r0, r1, g0, g1, s0, s1)

        return carry

    lax.fori_loop(0, npair, body, 0)

    @pl.when((npair - 1) % 2 == 0)
    def _():
        wait_scatter(r0, s0)
        wait_scatter(r1, s1)

    @pl.when((npair - 1) % 2 == 1)
    def _():
        wait_scatter(r2, s2)
        wait_scatter(r3, s3)

    plsc.subcore_barrier()
    pltpu.sync_copy(acc.at[pl.ds(s * RPT, RPT)], out.at[c, pl.ds(s * RPT, RPT)])


def _dinv_col(t):
    cols = [t[:, r:r + 1] for r in range(NRP // C)]
    return jnp.concatenate(cols, axis=0)[:N, :]


def _pre_body(degp_ref, x_ref, w1_ref, hp_ref, t_ref):
    d = degp_ref[0] + degp_ref[1] + 1.0
    dv = lax.rsqrt(d)
    t = jnp.transpose(dv)
    t_ref[...] = t
    h = jnp.dot(x_ref[...], w1_ref[...], preferred_element_type=jnp.float32)
    hp_ref[...] = h * _dinv_col(t)


_pre_call = pl.pallas_call(
    _pre_body,
    out_shape=(
        jax.ShapeDtypeStruct((N, H), jnp.float32),
        jax.ShapeDtypeStruct((C, NRP // C), jnp.float32),
    ),
)


def _mid_body(agg_ref, hp1_ref, t_ref, b1_ref, w2_ref, hp2_ref):
    dcol = _dinv_col(t_ref[...])
    a = agg_ref[0, :N, :] + agg_ref[1, :N, :]
    z1 = jnp.maximum(dcol * (a + hp1_ref[...]) + b1_ref[...], 0.0)
    hp2_ref[...] = jnp.dot(z1, w2_ref[...], preferred_element_type=jnp.float32) * dcol


_mid_call = pl.pallas_call(
    _mid_body,
    out_shape=jax.ShapeDtypeStruct((N, H), jnp.float32),
)


def _post_body(agg_ref, hp2_ref, t_ref, b2_ref, wl_ref, bl_ref, y_ref):
    dcol = _dinv_col(t_ref[...])
    a = agg_ref[0, :N, :] + agg_ref[1, :N, :]
    z2 = jnp.maximum(dcol * (a + hp2_ref[...]) + b2_ref[...], 0.0)
    y_ref[...] = jnp.dot(z2, wl_ref[...], preferred_element_type=jnp.float32) + bl_ref[...]


_post_call = pl.pallas_call(
    _post_body,
    out_shape=jax.ShapeDtypeStruct((N, 1), jnp.float32),
)


def kernel(x, edge_index, W1, b1, W2, b2, Wl, bl):
    ei3 = edge_index.reshape(2, NCHUNK, C)
    degp = _deg_kernel(ei3).reshape(NC, NRP // C, C)
    hp1, t = _pre_call(degp, x, W1)
    agg1 = _agg_kernel(hp1, ei3)
    hp2 = _mid_call(agg1, hp1, t, b1.reshape(1, H), W2)
    agg2 = _agg_kernel(hp2, ei3)
    y = _post_call(agg2, hp2, t, b2.reshape(1, H), Wl, bl.reshape(1, 1))
    return y
